# Initial kernel scaffold; baseline (speedup 1.0000x reference)
import jax
import jax.numpy as jnp
from jax.experimental import pallas as pl


def kernel(x, edge_index, edge_weights, W1, b1, W2, b2):
    raise NotImplementedError("write your pallas kernel here")


# speedup vs baseline: 9.3158x; 9.3158x over previous
import functools

import jax
import jax.numpy as jnp
from jax import lax
from jax.experimental import pallas as pl
from jax.experimental.pallas import tpu as pltpu
from jax.experimental.pallas import tpu_sc as plsc

N_NODES = 10000
D_IN = 128
D_HID = 256
HALF = D_HID // 2
N_CLS = 32

NUM_CORES = 2
NUM_SUBCORES = 16
CHUNK = 128
CHUNKS_PER_TILE = 160
E_PAD = NUM_SUBCORES * CHUNKS_PER_TILE * CHUNK
DEG_PER_TILE = E_PAD // (NUM_CORES * NUM_SUBCORES)
STRIPE = 640
LAST_STRIPE = N_NODES - (NUM_SUBCORES - 1) * STRIPE

_MESH = plsc.VectorSubcoreMesh(core_axis_name="c", subcore_axis_name="s")
_SC_PARAMS = pltpu.CompilerParams(needs_layout_passes=False)


@functools.partial(
    pl.kernel,
    out_type=jax.ShapeDtypeStruct((NUM_CORES * NUM_SUBCORES, N_NODES),
                                  jnp.float32),
    mesh=_MESH,
    compiler_params=_SC_PARAMS,
    scratch_types=[
        pltpu.VMEM((DEG_PER_TILE,), jnp.int32),
        pltpu.VMEM((DEG_PER_TILE,), jnp.float32),
        pltpu.VMEM((N_NODES,), jnp.float32),
    ],
)
def _sc_degree(col_hbm, ew_hbm, degp_hbm, col_v, ew_v, acc_v):
    c = lax.axis_index("c")
    s = lax.axis_index("s")
    wid = c * NUM_SUBCORES + s

    pltpu.sync_copy(col_hbm.at[wid], col_v)
    pltpu.sync_copy(ew_hbm.at[wid], ew_v)

    zero16 = jnp.zeros((16,), jnp.float32)

    def zero_body(i, carry):
        acc_v[pl.ds(i * 16, 16)] = zero16
        return carry

    lax.fori_loop(0, N_NODES // 16, zero_body, 0)

    def edge_body(j, carry):
        idx16 = col_v[pl.ds(j * 16, 16)]
        w16 = ew_v[pl.ds(j * 16, 16)]
        plsc.addupdate_scatter(acc_v, [idx16], w16)
        return carry

    lax.fori_loop(0, DEG_PER_TILE // 16, edge_body, 0)

    pltpu.sync_copy(acc_v, degp_hbm.at[wid])


@functools.partial(
    pl.kernel,
    out_type=(
        jax.ShapeDtypeStruct((N_NODES, HALF), jnp.float32),
        jax.ShapeDtypeStruct((N_NODES, HALF), jnp.float32),
    ),
    mesh=_MESH,
    compiler_params=_SC_PARAMS,
    scratch_types=[
        pltpu.VMEM((2, CHUNK), jnp.int32),
        pltpu.VMEM((CHUNKS_PER_TILE, CHUNK), jnp.float32),
        pltpu.VMEM((CHUNK, HALF), jnp.float32),
        pltpu.VMEM_SHARED((N_NODES, HALF), jnp.float32),
        pltpu.SemaphoreType.DMA,
    ],
)
def _sc_scatter(rc_hbm, ew_hbm, hcat_hbm, zero_hbm, o0_hbm, o1_hbm,
                rc_b, ew_v, rows_v, acc, sem):
    c = lax.axis_index("c")
    s = lax.axis_index("s")
    row_base = c * N_NODES

    def striped(fn):
        @pl.when(s < NUM_SUBCORES - 1)
        def _():
            off = pl.multiple_of(s * STRIPE, 8)
            fn(pl.ds(off, STRIPE))

        @pl.when(s == NUM_SUBCORES - 1)
        def _():
            fn(pl.ds((NUM_SUBCORES - 1) * STRIPE, LAST_STRIPE))

    striped(lambda sl: pltpu.sync_copy(zero_hbm.at[sl], acc.at[sl]))
    pltpu.sync_copy(ew_hbm.at[s], ew_v)
    plsc.subcore_barrier()

    def chunk_body(j, carry):
        pltpu.sync_copy(rc_hbm.at[s, j], rc_b)
        for q in range(CHUNK // 16):
            sl = pl.ds(q * 16, 16)
            rc_b[0, sl] = rc_b[0, sl] + row_base
        pltpu.async_copy(hcat_hbm.at[rc_b.at[0]], rows_v, sem).wait()

        def grp_body(q, cy):
            w16 = ew_v[j, pl.ds(q * 16, 16)]
            for e16 in range(16):
                w = w16[e16]
                e = q * 16 + e16
                for g in range(HALF // 16):
                    sl = pl.ds(g * 16, 16)
                    rows_v[e, sl] = rows_v[e, sl] * w
            return cy

        lax.fori_loop(0, CHUNK // 16, grp_body, 0)

        pltpu.sync_copy(rows_v, acc.at[rc_b.at[1]], add=True)
        return carry

    lax.fori_loop(0, CHUNKS_PER_TILE, chunk_body, 0)
    plsc.subcore_barrier()

    @pl.when(c == 0)
    def _():
        striped(lambda sl: pltpu.sync_copy(acc.at[sl], o0_hbm.at[sl]))

    @pl.when(c == 1)
    def _():
        striped(lambda sl: pltpu.sync_copy(acc.at[sl], o1_hbm.at[sl]))


BLK = 2000


def _dinv_from_partials(degp_ref):
    deg = jnp.sum(degp_ref[0], axis=0) + 1.0
    return jnp.where(deg > 0, lax.rsqrt(deg), 0.0)


def _tc_matmul_body(degp_ref, x_ref, w1_ref, h3_ref):
    dinv = _dinv_from_partials(degp_ref)
    h = jnp.dot(x_ref[...], w1_ref[...], preferred_element_type=jnp.float32)
    h = h * dinv[:, None]
    h3_ref[0] = h[:, :HALF]
    h3_ref[1] = h[:, HALF:]


def _tc_matmul(degp, x, w1):
    return pl.pallas_call(
        _tc_matmul_body,
        grid=(N_NODES // BLK,),
        in_specs=[
            pl.BlockSpec((1, NUM_CORES * NUM_SUBCORES, BLK),
                         lambda i: (i, 0, 0)),
            pl.BlockSpec((BLK, D_IN), lambda i: (i, 0)),
            pl.BlockSpec((D_IN, D_HID), lambda i: (0, 0)),
        ],
        out_specs=pl.BlockSpec((2, BLK, HALF), lambda i: (0, i, 0)),
        out_shape=jax.ShapeDtypeStruct((2, N_NODES, HALF), jnp.float32),
    )(degp, x, w1)


def _tc_post_body(degp_ref, s0_ref, s1_ref, h0_ref, h1_ref, b1_ref, w2_ref,
                  b2_ref, out_ref):
    dinv = _dinv_from_partials(degp_ref)[:, None]
    b1 = b1_ref[...]
    u0 = (s0_ref[...] + h0_ref[0]) * dinv + b1[:, :HALF]
    u1 = (s1_ref[...] + h1_ref[0]) * dinv + b1[:, HALF:]
    n2 = (jnp.sum(u0 * u0, axis=1, keepdims=True)
          + jnp.sum(u1 * u1, axis=1, keepdims=True))
    r = 1.0 / jnp.maximum(jnp.sqrt(n2), 1e-12)
    a0 = jnp.maximum(u0 * r, 0.0)
    a1 = jnp.maximum(u1 * r, 0.0)
    w2 = w2_ref[...]
    out_ref[...] = (
        jnp.dot(a0, w2[:HALF, :], preferred_element_type=jnp.float32)
        + jnp.dot(a1, w2[HALF:, :], preferred_element_type=jnp.float32)
        + b2_ref[...])


def _tc_post(degp, s0, s1, h3, b1, w2, b2):
    return pl.pallas_call(
        _tc_post_body,
        grid=(N_NODES // BLK,),
        in_specs=[
            pl.BlockSpec((1, NUM_CORES * NUM_SUBCORES, BLK),
                         lambda i: (i, 0, 0)),
            pl.BlockSpec((BLK, HALF), lambda i: (i, 0)),
            pl.BlockSpec((BLK, HALF), lambda i: (i, 0)),
            pl.BlockSpec((1, BLK, HALF), lambda i: (0, i, 0)),
            pl.BlockSpec((1, BLK, HALF), lambda i: (1, i, 0)),
            pl.BlockSpec((1, D_HID), lambda i: (0, 0)),
            pl.BlockSpec((D_HID, N_CLS), lambda i: (0, 0)),
            pl.BlockSpec((1, N_CLS), lambda i: (0, 0)),
        ],
        out_specs=pl.BlockSpec((BLK, N_CLS), lambda i: (i, 0)),
        out_shape=jax.ShapeDtypeStruct((N_NODES, N_CLS), jnp.float32),
    )(degp, s0, s1, h3, h3, b1, w2, b2)


def kernel(x, edge_index, edge_weights, W1, b1, W2, b2):
    e = edge_weights.shape[0]
    pad = E_PAD - e
    row = edge_index[0].astype(jnp.int32)
    col = edge_index[1].astype(jnp.int32)
    zi = jnp.zeros((pad,), jnp.int32)
    zf = jnp.zeros((pad,), jnp.float32)
    row_p = jnp.concatenate([row, zi])
    col_p = jnp.concatenate([col, zi])
    ew_p = jnp.concatenate([edge_weights.astype(jnp.float32), zf])

    col2 = col_p.reshape(NUM_CORES * NUM_SUBCORES, DEG_PER_TILE)
    ew2 = ew_p.reshape(NUM_CORES * NUM_SUBCORES, DEG_PER_TILE)
    rc3 = jnp.stack(
        [row_p.reshape(NUM_SUBCORES, CHUNKS_PER_TILE, CHUNK),
         col_p.reshape(NUM_SUBCORES, CHUNKS_PER_TILE, CHUNK)], axis=2)
    ew3 = ew_p.reshape(NUM_SUBCORES, CHUNKS_PER_TILE, CHUNK)

    degp = _sc_degree(col2, ew2)
    degp5 = degp.reshape(NUM_CORES * NUM_SUBCORES, N_NODES // BLK,
                         BLK).swapaxes(0, 1)
    h3 = _tc_matmul(degp5, x, W1)
    hcat = h3.reshape(NUM_CORES * N_NODES, HALF)
    zeros = jnp.zeros((N_NODES, HALF), jnp.float32)
    s0, s1 = _sc_scatter(rc3, ew3, hcat, zeros)
    return _tc_post(degp5, s0, s1, h3,
                    b1.reshape(1, D_HID).astype(jnp.float32),
                    W2, b2.reshape(1, N_CLS).astype(jnp.float32))


# speedup vs baseline: 13.3245x; 1.4303x over previous
import functools

import jax
import jax.numpy as jnp
from jax import lax
from jax.experimental import pallas as pl
from jax.experimental.pallas import tpu as pltpu
from jax.experimental.pallas import tpu_sc as plsc

N_NODES = 10000
D_IN = 128
D_HID = 256
HALF = D_HID // 2
N_CLS = 32

NUM_CORES = 2
NUM_SUBCORES = 16
CHUNK = 128
CHUNKS_PER_TILE = 160
E_PAD = NUM_SUBCORES * CHUNKS_PER_TILE * CHUNK
DEG_PER_TILE = E_PAD // (NUM_CORES * NUM_SUBCORES)
STRIPE = 640
LAST_STRIPE = N_NODES - (NUM_SUBCORES - 1) * STRIPE

_MESH = plsc.VectorSubcoreMesh(core_axis_name="c", subcore_axis_name="s")
_SC_PARAMS = pltpu.CompilerParams(needs_layout_passes=False)


@functools.partial(
    pl.kernel,
    out_type=jax.ShapeDtypeStruct((NUM_CORES * NUM_SUBCORES, N_NODES),
                                  jnp.float32),
    mesh=_MESH,
    compiler_params=_SC_PARAMS,
    scratch_types=[
        pltpu.VMEM((DEG_PER_TILE,), jnp.int32),
        pltpu.VMEM((DEG_PER_TILE,), jnp.float32),
        pltpu.VMEM((N_NODES,), jnp.float32),
    ],
)
def _sc_degree(col_hbm, ew_hbm, degp_hbm, col_v, ew_v, acc_v):
    c = lax.axis_index("c")
    s = lax.axis_index("s")
    wid = c * NUM_SUBCORES + s

    pltpu.sync_copy(col_hbm.at[wid], col_v)
    pltpu.sync_copy(ew_hbm.at[wid], ew_v)

    zero16 = jnp.zeros((16,), jnp.float32)

    def zero_body(i, carry):
        acc_v[pl.ds(i * 16, 16)] = zero16
        return carry

    lax.fori_loop(0, N_NODES // 16, zero_body, 0)

    def edge_body(j, carry):
        idx16 = col_v[pl.ds(j * 16, 16)]
        w16 = ew_v[pl.ds(j * 16, 16)]
        plsc.addupdate_scatter(acc_v, [idx16], w16)
        return carry

    lax.fori_loop(0, DEG_PER_TILE // 16, edge_body, 0)

    pltpu.sync_copy(acc_v, degp_hbm.at[wid])


@functools.partial(
    pl.kernel,
    out_type=(
        jax.ShapeDtypeStruct((N_NODES, HALF), jnp.float32),
        jax.ShapeDtypeStruct((N_NODES, HALF), jnp.float32),
    ),
    mesh=_MESH,
    compiler_params=_SC_PARAMS,
    scratch_types=[
        pltpu.VMEM((3, CHUNK), jnp.int32),
        pltpu.VMEM((3, CHUNK), jnp.int32),
        pltpu.VMEM((3, CHUNK), jnp.int32),
        pltpu.VMEM((3, CHUNK), jnp.int32),
        pltpu.VMEM((CHUNK, HALF), jnp.float32),
        pltpu.VMEM((CHUNK, HALF), jnp.float32),
        pltpu.VMEM_SHARED((N_NODES, HALF), jnp.float32),
        pltpu.SemaphoreType.DMA,
        pltpu.SemaphoreType.DMA,
        pltpu.SemaphoreType.DMA,
        pltpu.SemaphoreType.DMA,
    ],
)
def _sc_scatter(rcw_hbm, hcat_hbm, zero_hbm, o0_hbm, o1_hbm,
                rcw0, rcw1, rcw2, rcw3, rows0, rows1, acc,
                sg0, sg1, ss0, ss1):
    c = lax.axis_index("c")
    s = lax.axis_index("s")
    row_base = c * N_NODES
    rcw = (rcw0, rcw1, rcw2, rcw3)
    rows = (rows0, rows1)
    sg = (sg0, sg1)
    ss = (ss0, ss1)

    def striped(fn):
        @pl.when(s < NUM_SUBCORES - 1)
        def _():
            off = pl.multiple_of(s * STRIPE, 8)
            fn(pl.ds(off, STRIPE))

        @pl.when(s == NUM_SUBCORES - 1)
        def _():
            fn(pl.ds((NUM_SUBCORES - 1) * STRIPE, LAST_STRIPE))

    def fetch_idx(j, t):
        r = rcw[t]
        pltpu.sync_copy(rcw_hbm.at[s, j], r)
        for q in range(CHUNK // 16):
            sl = pl.ds(q * 16, 16)
            r[0, sl] = r[0, sl] + row_base

    def start_gather(t, b):
        pltpu.async_copy(hcat_hbm.at[rcw[t].at[0]], rows[b], sg[b])

    def wait_dma(sem, b):
        pltpu.make_async_copy(zero_hbm.at[pl.ds(0, CHUNK)], rows[b],
                              sem).wait()

    def scale(t, b):
        def grp_body(q, cy):
            w16 = plsc.bitcast(rcw[t][2, pl.ds(q * 16, 16)], jnp.float32)
            for e16 in range(16):
                w = w16[e16]
                e = q * 16 + e16
                for g in range(HALF // 16):
                    sl = pl.ds(g * 16, 16)
                    rows[b][e, sl] = rows[b][e, sl] * w
            return cy

        lax.fori_loop(0, CHUNK // 16, grp_body, 0)

    def start_scatter(t, b):
        pltpu.async_copy(rows[b], acc.at[rcw[t].at[1]], ss[b], add=True)

    striped(lambda sl: pltpu.sync_copy(zero_hbm.at[sl], acc.at[sl]))
    plsc.subcore_barrier()

    fetch_idx(0, 0)
    fetch_idx(1, 1)
    fetch_idx(2, 2)
    start_gather(0, 0)
    start_gather(1, 1)
    wait_dma(sg[0], 0)
    scale(0, 0)
    start_scatter(0, 0)
    fetch_idx(3, 3)
    for k, base in ((1, 4), (2, 5), (3, 6)):
        t, b, nb = k & 3, k & 1, 1 - (k & 1)
        wait_dma(ss[nb], nb)
        start_gather((k + 1) & 3, nb)
        wait_dma(sg[b], b)
        scale(t, b)
        start_scatter(t, b)
        fetch_idx(base, base & 3)

    def pipe_body(k4, carry):
        k0 = 4 * k4 + 4
        for m in range(4):
            b, nb = m & 1, 1 - (m & 1)
            wait_dma(ss[nb], nb)
            start_gather((m + 1) & 3, nb)
            wait_dma(sg[b], b)
            scale(m, b)
            start_scatter(m, b)
            fetch_idx(k0 + m + 3, (m + 3) & 3)
        return carry

    lax.fori_loop(0, (CHUNKS_PER_TILE - 8) // 4, pipe_body, 0)

    for k in (156, 157, 158, 159):
        t, b, nb = k & 3, k & 1, 1 - (k & 1)
        wait_dma(ss[nb], nb)
        if k < 159:
            start_gather((k + 1) & 3, nb)
        wait_dma(sg[b], b)
        scale(t, b)
        start_scatter(t, b)
        if k == 156:
            fetch_idx(159, 3)
    wait_dma(ss[1], 1)
    plsc.subcore_barrier()

    @pl.when(c == 0)
    def _():
        striped(lambda sl: pltpu.sync_copy(acc.at[sl], o0_hbm.at[sl]))

    @pl.when(c == 1)
    def _():
        striped(lambda sl: pltpu.sync_copy(acc.at[sl], o1_hbm.at[sl]))


BLK = 2000


def _dinv_from_partials(degp_ref):
    deg = jnp.sum(degp_ref[0], axis=0) + 1.0
    return jnp.where(deg > 0, lax.rsqrt(deg), 0.0)


def _tc_matmul_body(degp_ref, x_ref, w1_ref, h3_ref):
    dinv = _dinv_from_partials(degp_ref)
    h = jnp.dot(x_ref[...], w1_ref[...], preferred_element_type=jnp.float32)
    h = h * dinv[:, None]
    h3_ref[0] = h[:, :HALF]
    h3_ref[1] = h[:, HALF:]


def _tc_matmul(degp, x, w1):
    return pl.pallas_call(
        _tc_matmul_body,
        grid=(N_NODES // BLK,),
        in_specs=[
            pl.BlockSpec((1, NUM_CORES * NUM_SUBCORES, BLK),
                         lambda i: (i, 0, 0)),
            pl.BlockSpec((BLK, D_IN), lambda i: (i, 0)),
            pl.BlockSpec((D_IN, D_HID), lambda i: (0, 0)),
        ],
        out_specs=pl.BlockSpec((2, BLK, HALF), lambda i: (0, i, 0)),
        out_shape=jax.ShapeDtypeStruct((2, N_NODES, HALF), jnp.float32),
    )(degp, x, w1)


def _tc_post_body(degp_ref, s0_ref, s1_ref, h0_ref, h1_ref, b1_ref, w2_ref,
                  b2_ref, out_ref):
    dinv = _dinv_from_partials(degp_ref)[:, None]
    b1 = b1_ref[...]
    u0 = (s0_ref[...] + h0_ref[0]) * dinv + b1[:, :HALF]
    u1 = (s1_ref[...] + h1_ref[0]) * dinv + b1[:, HALF:]
    n2 = (jnp.sum(u0 * u0, axis=1, keepdims=True)
          + jnp.sum(u1 * u1, axis=1, keepdims=True))
    r = 1.0 / jnp.maximum(jnp.sqrt(n2), 1e-12)
    a0 = jnp.maximum(u0 * r, 0.0)
    a1 = jnp.maximum(u1 * r, 0.0)
    w2 = w2_ref[...]
    out_ref[...] = (
        jnp.dot(a0, w2[:HALF, :], preferred_element_type=jnp.float32)
        + jnp.dot(a1, w2[HALF:, :], preferred_element_type=jnp.float32)
        + b2_ref[...])


def _tc_post(degp, s0, s1, h3, b1, w2, b2):
    return pl.pallas_call(
        _tc_post_body,
        grid=(N_NODES // BLK,),
        in_specs=[
            pl.BlockSpec((1, NUM_CORES * NUM_SUBCORES, BLK),
                         lambda i: (i, 0, 0)),
            pl.BlockSpec((BLK, HALF), lambda i: (i, 0)),
            pl.BlockSpec((BLK, HALF), lambda i: (i, 0)),
            pl.BlockSpec((1, BLK, HALF), lambda i: (0, i, 0)),
            pl.BlockSpec((1, BLK, HALF), lambda i: (1, i, 0)),
            pl.BlockSpec((1, D_HID), lambda i: (0, 0)),
            pl.BlockSpec((D_HID, N_CLS), lambda i: (0, 0)),
            pl.BlockSpec((1, N_CLS), lambda i: (0, 0)),
        ],
        out_specs=pl.BlockSpec((BLK, N_CLS), lambda i: (i, 0)),
        out_shape=jax.ShapeDtypeStruct((N_NODES, N_CLS), jnp.float32),
    )(degp, s0, s1, h3, h3, b1, w2, b2)


def kernel(x, edge_index, edge_weights, W1, b1, W2, b2):
    e = edge_weights.shape[0]
    pad = E_PAD - e
    row = edge_index[0].astype(jnp.int32)
    col = edge_index[1].astype(jnp.int32)
    zi = jnp.zeros((pad,), jnp.int32)
    zf = jnp.zeros((pad,), jnp.float32)
    row_p = jnp.concatenate([row, zi])
    col_p = jnp.concatenate([col, zi])
    ew_p = jnp.concatenate([edge_weights.astype(jnp.float32), zf])

    col2 = col_p.reshape(NUM_CORES * NUM_SUBCORES, DEG_PER_TILE)
    ew2 = ew_p.reshape(NUM_CORES * NUM_SUBCORES, DEG_PER_TILE)
    ew_bits = lax.bitcast_convert_type(ew_p, jnp.int32)
    rcw3 = jnp.stack(
        [row_p.reshape(NUM_SUBCORES, CHUNKS_PER_TILE, CHUNK),
         col_p.reshape(NUM_SUBCORES, CHUNKS_PER_TILE, CHUNK),
         ew_bits.reshape(NUM_SUBCORES, CHUNKS_PER_TILE, CHUNK)], axis=2)

    degp = _sc_degree(col2, ew2)
    degp5 = degp.reshape(NUM_CORES * NUM_SUBCORES, N_NODES // BLK,
                         BLK).swapaxes(0, 1)
    h3 = _tc_matmul(degp5, x, W1)
    hcat = h3.reshape(NUM_CORES * N_NODES, HALF)
    zeros = jnp.zeros((N_NODES, HALF), jnp.float32)
    s0, s1 = _sc_scatter(rcw3, hcat, zeros)
    return _tc_post(degp5, s0, s1, h3,
                    b1.reshape(1, D_HID).astype(jnp.float32),
                    W2, b2.reshape(1, N_CLS).astype(jnp.float32))


# speedup vs baseline: 13.3881x; 1.0048x over previous
import functools

import jax
import jax.numpy as jnp
from jax import lax
from jax.experimental import pallas as pl
from jax.experimental.pallas import tpu as pltpu
from jax.experimental.pallas import tpu_sc as plsc

N_NODES = 10000
D_IN = 128
D_HID = 256
HALF = D_HID // 2
N_CLS = 32

NUM_CORES = 2
NUM_SUBCORES = 16
CHUNK = 128
CHUNKS_PER_TILE = 160
E_PAD = NUM_SUBCORES * CHUNKS_PER_TILE * CHUNK
DEG_PER_TILE = E_PAD // (NUM_CORES * NUM_SUBCORES)
STRIPE = 640
LAST_STRIPE = N_NODES - (NUM_SUBCORES - 1) * STRIPE

_MESH = plsc.VectorSubcoreMesh(core_axis_name="c", subcore_axis_name="s")
_SC_PARAMS = pltpu.CompilerParams(needs_layout_passes=False)


@functools.partial(
    pl.kernel,
    out_type=jax.ShapeDtypeStruct((NUM_CORES * NUM_SUBCORES, N_NODES),
                                  jnp.float32),
    mesh=_MESH,
    compiler_params=_SC_PARAMS,
    scratch_types=[
        pltpu.VMEM((DEG_PER_TILE,), jnp.int32),
        pltpu.VMEM((DEG_PER_TILE,), jnp.float32),
        pltpu.VMEM((N_NODES,), jnp.float32),
    ],
)
def _sc_degree(col_hbm, ew_hbm, degp_hbm, col_v, ew_v, acc_v):
    c = lax.axis_index("c")
    s = lax.axis_index("s")
    wid = c * NUM_SUBCORES + s

    pltpu.sync_copy(col_hbm.at[wid], col_v)
    pltpu.sync_copy(ew_hbm.at[wid], ew_v)

    zero16 = jnp.zeros((16,), jnp.float32)

    def zero_body(i, carry):
        acc_v[pl.ds(i * 16, 16)] = zero16
        return carry

    lax.fori_loop(0, N_NODES // 16, zero_body, 0)

    def edge_body(j, carry):
        idx16 = col_v[pl.ds(j * 16, 16)]
        w16 = ew_v[pl.ds(j * 16, 16)]
        plsc.addupdate_scatter(acc_v, [idx16], w16)
        return carry

    lax.fori_loop(0, DEG_PER_TILE // 16, edge_body, 0)

    pltpu.sync_copy(acc_v, degp_hbm.at[wid])


@functools.partial(
    pl.kernel,
    out_type=(
        jax.ShapeDtypeStruct((N_NODES, HALF), jnp.float32),
        jax.ShapeDtypeStruct((N_NODES, HALF), jnp.float32),
    ),
    mesh=_MESH,
    compiler_params=_SC_PARAMS,
    scratch_types=[
        pltpu.VMEM((3, CHUNK), jnp.int32),
        pltpu.VMEM((3, CHUNK), jnp.int32),
        pltpu.VMEM((3, CHUNK), jnp.int32),
        pltpu.VMEM((3, CHUNK), jnp.int32),
        pltpu.VMEM((CHUNK, HALF), jnp.float32),
        pltpu.VMEM((CHUNK, HALF), jnp.float32),
        pltpu.VMEM_SHARED((N_NODES, HALF), jnp.float32),
        pltpu.SemaphoreType.DMA,
        pltpu.SemaphoreType.DMA,
        pltpu.SemaphoreType.DMA,
        pltpu.SemaphoreType.DMA,
        pltpu.SemaphoreType.DMA,
        pltpu.SemaphoreType.DMA,
        pltpu.SemaphoreType.DMA,
        pltpu.SemaphoreType.DMA,
    ],
)
def _sc_scatter(rcw_hbm, hcat_hbm, zero_hbm, o0_hbm, o1_hbm,
                rcw0, rcw1, rcw2, rcw3, rows0, rows1, acc,
                sg0, sg1, ss0, ss1, sf0, sf1, sf2, sf3):
    c = lax.axis_index("c")
    s = lax.axis_index("s")
    row_base = c * N_NODES
    rcw = (rcw0, rcw1, rcw2, rcw3)
    rows = (rows0, rows1)
    sg = (sg0, sg1)
    ss = (ss0, ss1)
    sf = (sf0, sf1, sf2, sf3)

    def striped(fn):
        @pl.when(s < NUM_SUBCORES - 1)
        def _():
            off = pl.multiple_of(s * STRIPE, 8)
            fn(pl.ds(off, STRIPE))

        @pl.when(s == NUM_SUBCORES - 1)
        def _():
            fn(pl.ds((NUM_SUBCORES - 1) * STRIPE, LAST_STRIPE))

    def fetch_idx(j, t):
        pltpu.async_copy(rcw_hbm.at[s, j], rcw[t], sf[t])

    def start_gather(t, b):
        r = rcw[t]
        pltpu.make_async_copy(rcw_hbm.at[s, 0], r, sf[t]).wait()
        for q in range(CHUNK // 16):
            sl = pl.ds(q * 16, 16)
            r[0, sl] = r[0, sl] + row_base
        pltpu.async_copy(hcat_hbm.at[r.at[0]], rows[b], sg[b])

    def wait_dma(sem, b):
        pltpu.make_async_copy(zero_hbm.at[pl.ds(0, CHUNK)], rows[b],
                              sem).wait()

    def scale(t, b):
        def grp_body(q, cy):
            w16 = plsc.bitcast(rcw[t][2, pl.ds(q * 16, 16)], jnp.float32)
            for e16 in range(16):
                w = w16[e16]
                e = q * 16 + e16
                for g in range(HALF // 16):
                    sl = pl.ds(g * 16, 16)
                    rows[b][e, sl] = rows[b][e, sl] * w
            return cy

        lax.fori_loop(0, CHUNK // 16, grp_body, 0)

    def start_scatter(t, b):
        pltpu.async_copy(rows[b], acc.at[rcw[t].at[1]], ss[b], add=True)

    striped(lambda sl: pltpu.sync_copy(zero_hbm.at[sl], acc.at[sl]))
    plsc.subcore_barrier()

    fetch_idx(0, 0)
    fetch_idx(1, 1)
    fetch_idx(2, 2)
    start_gather(0, 0)
    start_gather(1, 1)
    wait_dma(sg[0], 0)
    scale(0, 0)
    start_scatter(0, 0)
    fetch_idx(3, 3)
    for k, base in ((1, 4), (2, 5), (3, 6)):
        t, b, nb = k & 3, k & 1, 1 - (k & 1)
        wait_dma(ss[nb], nb)
        start_gather((k + 1) & 3, nb)
        wait_dma(sg[b], b)
        scale(t, b)
        start_scatter(t, b)
        fetch_idx(base, base & 3)

    def pipe_body(k4, carry):
        k0 = 4 * k4 + 4
        for m in range(4):
            b, nb = m & 1, 1 - (m & 1)
            wait_dma(ss[nb], nb)
            start_gather((m + 1) & 3, nb)
            wait_dma(sg[b], b)
            scale(m, b)
            start_scatter(m, b)
            fetch_idx(k0 + m + 3, (m + 3) & 3)
        return carry

    lax.fori_loop(0, (CHUNKS_PER_TILE - 8) // 4, pipe_body, 0)

    for k in (156, 157, 158, 159):
        t, b, nb = k & 3, k & 1, 1 - (k & 1)
        wait_dma(ss[nb], nb)
        if k < 159:
            start_gather((k + 1) & 3, nb)
        wait_dma(sg[b], b)
        scale(t, b)
        start_scatter(t, b)
        if k == 156:
            fetch_idx(159, 3)
    wait_dma(ss[1], 1)
    plsc.subcore_barrier()

    @pl.when(c == 0)
    def _():
        striped(lambda sl: pltpu.sync_copy(acc.at[sl], o0_hbm.at[sl]))

    @pl.when(c == 1)
    def _():
        striped(lambda sl: pltpu.sync_copy(acc.at[sl], o1_hbm.at[sl]))


BLK = 2000


def _dinv_from_partials(degp_ref):
    deg = jnp.sum(degp_ref[0], axis=0) + 1.0
    return jnp.where(deg > 0, lax.rsqrt(deg), 0.0)


def _tc_matmul_body(degp_ref, x_ref, w1_ref, h3_ref):
    dinv = _dinv_from_partials(degp_ref)
    h = jnp.dot(x_ref[...], w1_ref[...], preferred_element_type=jnp.float32)
    h = h * dinv[:, None]
    h3_ref[0] = h[:, :HALF]
    h3_ref[1] = h[:, HALF:]


def _tc_matmul(degp, x, w1):
    return pl.pallas_call(
        _tc_matmul_body,
        grid=(N_NODES // BLK,),
        in_specs=[
            pl.BlockSpec((1, NUM_CORES * NUM_SUBCORES, BLK),
                         lambda i: (i, 0, 0)),
            pl.BlockSpec((BLK, D_IN), lambda i: (i, 0)),
            pl.BlockSpec((D_IN, D_HID), lambda i: (0, 0)),
        ],
        out_specs=pl.BlockSpec((2, BLK, HALF), lambda i: (0, i, 0)),
        out_shape=jax.ShapeDtypeStruct((2, N_NODES, HALF), jnp.float32),
    )(degp, x, w1)


def _tc_post_body(degp_ref, s0_ref, s1_ref, h0_ref, h1_ref, b1_ref, w2_ref,
                  b2_ref, out_ref):
    dinv = _dinv_from_partials(degp_ref)[:, None]
    b1 = b1_ref[...]
    u0 = (s0_ref[...] + h0_ref[0]) * dinv + b1[:, :HALF]
    u1 = (s1_ref[...] + h1_ref[0]) * dinv + b1[:, HALF:]
    n2 = (jnp.sum(u0 * u0, axis=1, keepdims=True)
          + jnp.sum(u1 * u1, axis=1, keepdims=True))
    r = 1.0 / jnp.maximum(jnp.sqrt(n2), 1e-12)
    a0 = jnp.maximum(u0 * r, 0.0)
    a1 = jnp.maximum(u1 * r, 0.0)
    w2 = w2_ref[...]
    out_ref[...] = (
        jnp.dot(a0, w2[:HALF, :], preferred_element_type=jnp.float32)
        + jnp.dot(a1, w2[HALF:, :], preferred_element_type=jnp.float32)
        + b2_ref[...])


def _tc_post(degp, s0, s1, h3, b1, w2, b2):
    return pl.pallas_call(
        _tc_post_body,
        grid=(N_NODES // BLK,),
        in_specs=[
            pl.BlockSpec((1, NUM_CORES * NUM_SUBCORES, BLK),
                         lambda i: (i, 0, 0)),
            pl.BlockSpec((BLK, HALF), lambda i: (i, 0)),
            pl.BlockSpec((BLK, HALF), lambda i: (i, 0)),
            pl.BlockSpec((1, BLK, HALF), lambda i: (0, i, 0)),
            pl.BlockSpec((1, BLK, HALF), lambda i: (1, i, 0)),
            pl.BlockSpec((1, D_HID), lambda i: (0, 0)),
            pl.BlockSpec((D_HID, N_CLS), lambda i: (0, 0)),
            pl.BlockSpec((1, N_CLS), lambda i: (0, 0)),
        ],
        out_specs=pl.BlockSpec((BLK, N_CLS), lambda i: (i, 0)),
        out_shape=jax.ShapeDtypeStruct((N_NODES, N_CLS), jnp.float32),
    )(degp, s0, s1, h3, h3, b1, w2, b2)


def kernel(x, edge_index, edge_weights, W1, b1, W2, b2):
    e = edge_weights.shape[0]
    pad = E_PAD - e
    row = edge_index[0].astype(jnp.int32)
    col = edge_index[1].astype(jnp.int32)
    zi = jnp.zeros((pad,), jnp.int32)
    zf = jnp.zeros((pad,), jnp.float32)
    row_p = jnp.concatenate([row, zi])
    col_p = jnp.concatenate([col, zi])
    ew_p = jnp.concatenate([edge_weights.astype(jnp.float32), zf])

    col2 = col_p.reshape(NUM_CORES * NUM_SUBCORES, DEG_PER_TILE)
    ew2 = ew_p.reshape(NUM_CORES * NUM_SUBCORES, DEG_PER_TILE)
    ew_bits = lax.bitcast_convert_type(ew_p, jnp.int32)
    rcw3 = jnp.stack(
        [row_p.reshape(NUM_SUBCORES, CHUNKS_PER_TILE, CHUNK),
         col_p.reshape(NUM_SUBCORES, CHUNKS_PER_TILE, CHUNK),
         ew_bits.reshape(NUM_SUBCORES, CHUNKS_PER_TILE, CHUNK)], axis=2)

    degp = _sc_degree(col2, ew2)
    degp5 = degp.reshape(NUM_CORES * NUM_SUBCORES, N_NODES // BLK,
                         BLK).swapaxes(0, 1)
    h3 = _tc_matmul(degp5, x, W1)
    hcat = h3.reshape(NUM_CORES * N_NODES, HALF)
    zeros = jnp.zeros((N_NODES, HALF), jnp.float32)
    s0, s1 = _sc_scatter(rcw3, hcat, zeros)
    return _tc_post(degp5, s0, s1, h3,
                    b1.reshape(1, D_HID).astype(jnp.float32),
                    W2, b2.reshape(1, N_CLS).astype(jnp.float32))
